# baseline (speedup 1.0000x reference)
import jax
import jax.numpy as jnp
import numpy as np
from jax.experimental import pallas as pl

_N, _P, _D = 32, 210, 768
_H = _N // 2
_NEG_INF = float("-inf")
_OFFS = (0, -1, 1, -10, 10, -11, -9, 9, 11)
_HI = jax.lax.Precision.HIGHEST


def _build_constants():
    cpu = jax.local_devices(backend="cpu")[0]
    with jax.default_device(cpu):
        return _build_constants_impl()


def _build_constants_impl():
    base = jax.random.key(42)
    keys = jax.vmap(lambda c: jax.random.fold_in(base, c))(jnp.arange(1, 65))
    att_ids = jnp.asarray([16 * b + t for b in range(4) for t in range(8)])
    mut_ids = jnp.asarray([16 * b + 8 + t for b in range(4) for t in range(8)])
    att_keys = keys[att_ids]
    mut_keys = keys[mut_ids]
    att_perms = np.asarray(
        jax.vmap(lambda k: jax.random.permutation(k, _P))(att_keys))
    zs = np.asarray(
        jax.vmap(lambda k: jax.random.normal(k, (_P, _D), jnp.float32))(mut_keys))
    mut_perm_keys = jax.vmap(lambda k: jax.random.fold_in(k, 1))(mut_keys)
    mut_perms = np.asarray(
        jax.vmap(lambda k: jax.random.permutation(k, _P))(mut_perm_keys))

    r40 = att_perms[:, :40]
    gs = np.zeros((32, 40, _P), np.float32)
    gs[np.arange(32)[:, None], np.arange(40)[None, :], r40] = 1.0
    ps = np.ascontiguousarray(np.transpose(gs, (0, 2, 1)))
    ms = gs.sum(axis=1)[..., None]

    r20 = mut_perms[:, :20]
    g20 = np.zeros((32, 20, _P), np.float32)
    g20[np.arange(32)[:, None], np.arange(20)[None, :], r20] = 1.0
    p20 = np.ascontiguousarray(np.transpose(g20, (0, 2, 1)))
    m20 = g20.sum(axis=1)[..., None]
    zsel = zs[np.arange(32)[:, None], r20]

    return (
        gs.reshape(4, 8, 40, _P),
        ps.reshape(4, 8, _P, 40),
        ms.reshape(4, 8, _P, 1),
        g20.reshape(4, 8, 20, _P),
        p20.reshape(4, 8, _P, 20),
        m20.reshape(4, 8, _P, 1),
        zsel.reshape(4, 8, 20, _D),
    )


_CONSTS = _build_constants()


def _hdot(a, b):
    return jnp.dot(a, b, precision=_HI, preferred_element_type=jnp.float32)


def _tree_cols(e, n, op, pad_val):
    p2 = 1
    while p2 < n:
        p2 *= 2
    if p2 != n:
        pad = jnp.full((e.shape[0], p2 - n), pad_val, e.dtype)
        e = jnp.concatenate([e, pad], axis=1)
    stride = p2 // 2
    while stride >= 1:
        e = op(e[:, :stride], e[:, stride:2 * stride])
        stride //= 2
    return e


def _attention(a_row, b_row, g, p, m):
    scores = jax.lax.dot_general(
        a_row, b_row, (((1,), (1,)), ((), ())),
        preferred_element_type=jnp.float32)
    s_sel = _hdot(g, scores)

    mx = jnp.max(s_sel, axis=1, keepdims=True)
    e = jnp.exp(s_sel - mx)
    den = _tree_cols(e, _P, jnp.add, 0.0)
    w = e / den

    iota = jax.lax.broadcasted_iota(jnp.int32, (40, _P), 1)
    wmax = jnp.max(w, axis=1, keepdims=True)
    mp = jnp.min(jnp.where(w == wmax, iota, _P), axis=1, keepdims=True)
    col = mp % 10
    valids = (
        mp >= 0,
        col != 0,
        col != 9,
        mp > 9,
        mp < 200,
        (mp > 9) & (col > 0),
        (mp > 9) & (col < 9),
        (mp < 200) & (col > 0),
        (mp < 200) & (col < 9),
    )
    onehots, wvs = [], []
    for off, valid in zip(_OFFS, valids):
        idx = jnp.clip(mp + off, 0, _P - 1)
        oh = (iota == idx).astype(jnp.float32)
        onehots.append(oh)
        wv = jnp.sum(oh * w, axis=1, keepdims=True)
        wvs.append(jnp.where(valid, wv, _NEG_INF))
    wv9 = jnp.concatenate(wvs, axis=1)
    mx9 = _tree_cols(wv9, 9, jnp.maximum, _NEG_INF)
    e9 = jnp.exp(wv9 - mx9)
    den9 = _tree_cols(e9, 9, jnp.add, 0.0)
    wn = e9 / den9

    oh_all = jnp.concatenate(onehots, axis=0)
    fv_all = _hdot(oh_all, b_row)
    ff = fv_all[0:40, :] * wn[:, 0:1]
    for o in range(1, 9):
        ff = ff + fv_all[o * 40:(o + 1) * 40, :] * wn[:, o:o + 1]
    return jnp.where(m > 0, _hdot(p, ff), a_row)


def _body(a_ref, b_ref, gs_ref, ps_ref, ms_ref, g20_ref, p20_ref, m20_ref,
          z_ref, oa_ref, ob_ref):
    A = [a_ref[i] for i in range(4)]
    B = [b_ref[i] for i in range(4)]

    s6 = A[0] + A[1] + A[2] + B[0] + B[1] + B[2]
    mean = s6 / 6.0
    var = ((A[0] - mean) ** 2 + (A[1] - mean) ** 2 + (A[2] - mean) ** 2
           + (B[0] - mean) ** 2 + (B[1] - mean) ** 2 + (B[2] - mean) ** 2) / 5.0
    std = jnp.sqrt(var + 1e-8)

    def att(x, y, t):
        return _attention(x, y, gs_ref[0, t], ps_ref[0, t], ms_ref[0, t])

    A[0] = att(A[0], B[0], 0)
    B[0] = att(B[0], A[0], 1)
    B[1] = att(B[1], A[1], 2)
    A[1] = att(A[1], B[1], 3)
    A[2] = att(A[2], A[3], 4)
    A[3] = att(A[3], A[2], 5)
    B[2] = att(B[2], B[3], 6)
    B[3] = att(B[3], B[2], 7)

    order = ((A, 0), (B, 0), (A, 1), (B, 1), (A, 2), (B, 2), (A, 3), (B, 3))
    for mi, (half, i) in enumerate(order):
        pts = _hdot(g20_ref[0, mi], mean) + _hdot(g20_ref[0, mi], std) * z_ref[0, mi]
        half[i] = jnp.where(m20_ref[0, mi] > 0, _hdot(p20_ref[0, mi], pts), half[i])

    for i in range(4):
        oa_ref[i] = A[i]
        ob_ref[i] = B[i]


def kernel(inputss):
    first = inputss[:_H]
    second = inputss[_H:]
    gs, ps, ms, g20, p20, m20, zsel = _CONSTS
    row_spec = pl.BlockSpec((4, _P, _D), lambda b: (b, 0, 0))
    out_a, out_b = pl.pallas_call(
        _body,
        grid=(4,),
        in_specs=[
            row_spec,
            row_spec,
            pl.BlockSpec((1, 8, 40, _P), lambda b: (b, 0, 0, 0)),
            pl.BlockSpec((1, 8, _P, 40), lambda b: (b, 0, 0, 0)),
            pl.BlockSpec((1, 8, _P, 1), lambda b: (b, 0, 0, 0)),
            pl.BlockSpec((1, 8, 20, _P), lambda b: (b, 0, 0, 0)),
            pl.BlockSpec((1, 8, _P, 20), lambda b: (b, 0, 0, 0)),
            pl.BlockSpec((1, 8, _P, 1), lambda b: (b, 0, 0, 0)),
            pl.BlockSpec((1, 8, 20, _D), lambda b: (b, 0, 0, 0)),
        ],
        out_specs=[row_spec, row_spec],
        out_shape=[
            jax.ShapeDtypeStruct((_H, _P, _D), jnp.float32),
            jax.ShapeDtypeStruct((_H, _P, _D), jnp.float32),
        ],
    )(first, second, gs, ps, ms, g20, p20, m20, zsel)
    return jnp.concatenate([out_a, out_b], axis=0)


# speedup vs baseline: 1.5596x; 1.5596x over previous
import jax
import jax.numpy as jnp
import numpy as np
from jax.experimental import pallas as pl

_N, _P, _D = 32, 210, 768
_H = _N // 2
_NEG_INF = float("-inf")
_OFFS = (0, -1, 1, -10, 10, -11, -9, 9, 11)


def _build_constants():
    cpu = jax.local_devices(backend="cpu")[0]
    with jax.default_device(cpu):
        return _build_constants_impl()


def _build_constants_impl():
    base = jax.random.key(42)
    keys = jax.vmap(lambda c: jax.random.fold_in(base, c))(jnp.arange(1, 65))
    att_ids = jnp.asarray([16 * b + t for b in range(4) for t in range(8)])
    mut_ids = jnp.asarray([16 * b + 8 + t for b in range(4) for t in range(8)])
    att_keys = keys[att_ids]
    mut_keys = keys[mut_ids]
    att_perms = np.asarray(
        jax.vmap(lambda k: jax.random.permutation(k, _P))(att_keys))
    zs = np.asarray(
        jax.vmap(lambda k: jax.random.normal(k, (_P, _D), jnp.float32))(mut_keys))
    mut_perm_keys = jax.vmap(lambda k: jax.random.fold_in(k, 1))(mut_keys)
    mut_perms = np.asarray(
        jax.vmap(lambda k: jax.random.permutation(k, _P))(mut_perm_keys))

    r40 = att_perms[:, :40]
    gs = np.zeros((32, 40, _P), np.float32)
    gs[np.arange(32)[:, None], np.arange(40)[None, :], r40] = 1.0
    ps = np.ascontiguousarray(np.transpose(gs, (0, 2, 1)))
    ms = gs.sum(axis=1)[..., None]

    r20 = mut_perms[:, :20]
    g20 = np.zeros((32, 20, _P), np.float32)
    g20[np.arange(32)[:, None], np.arange(20)[None, :], r20] = 1.0
    p20 = np.ascontiguousarray(np.transpose(g20, (0, 2, 1)))
    m20 = g20.sum(axis=1)[..., None]
    zsel = zs[np.arange(32)[:, None], r20]

    return (
        gs.reshape(4, 8, 40, _P).astype(jnp.bfloat16),
        ps.reshape(4, 8, _P, 40).astype(jnp.bfloat16),
        ms.reshape(4, 8, _P, 1),
        g20.reshape(4, 8, 20, _P).astype(jnp.bfloat16),
        p20.reshape(4, 8, _P, 20).astype(jnp.bfloat16),
        m20.reshape(4, 8, _P, 1),
        zsel.reshape(4, 8, 20, _D),
    )


_CONSTS = _build_constants()


def _bdot(a_bf, b_bf):
    return jax.lax.dot_general(a_bf, b_bf, (((1,), (0,)), ((), ())),
                               preferred_element_type=jnp.float32)


def _split3(x):
    hi = x.astype(jnp.bfloat16)
    r1 = x - hi.astype(jnp.float32)
    lo1 = r1.astype(jnp.bfloat16)
    r2 = r1 - lo1.astype(jnp.float32)
    lo2 = r2.astype(jnp.bfloat16)
    return hi, lo1, lo2


def _exact_onehot_dot(oh_bf, x):
    hi, lo1, lo2 = _split3(x)
    return (_bdot(oh_bf, hi) + _bdot(oh_bf, lo1)) + _bdot(oh_bf, lo2)


def _tree_cols(e, n, op, pad_val):
    p2 = 1
    while p2 < n:
        p2 *= 2
    if p2 != n:
        pad = jnp.full((e.shape[0], p2 - n), pad_val, e.dtype)
        e = jnp.concatenate([e, pad], axis=1)
    stride = p2 // 2
    while stride >= 1:
        e = op(e[:, :stride], e[:, stride:2 * stride])
        stride //= 2
    return e


def _attention(a_row, b_row, g, p, m):
    scores = jax.lax.dot_general(
        a_row, b_row, (((1,), (1,)), ((), ())),
        preferred_element_type=jnp.float32)
    s_sel = _exact_onehot_dot(g, scores)

    mx = jnp.max(s_sel, axis=1, keepdims=True)
    e = jnp.exp(s_sel - mx)
    den = _tree_cols(e, _P, jnp.add, 0.0)
    w = e / den

    iota = jax.lax.broadcasted_iota(jnp.int32, (40, _P), 1)
    wmax = jnp.max(w, axis=1, keepdims=True)
    mp = jnp.min(jnp.where(w == wmax, iota, _P), axis=1, keepdims=True)
    col = mp % 10
    valids = (
        mp >= 0,
        col != 0,
        col != 9,
        mp > 9,
        mp < 200,
        (mp > 9) & (col > 0),
        (mp > 9) & (col < 9),
        (mp < 200) & (col > 0),
        (mp < 200) & (col < 9),
    )
    onehots, wvs = [], []
    for off, valid in zip(_OFFS, valids):
        idx = jnp.clip(mp + off, 0, _P - 1)
        oh = (iota == idx)
        onehots.append(oh.astype(jnp.bfloat16))
        wv = jnp.sum(oh.astype(jnp.float32) * w, axis=1, keepdims=True)
        wvs.append(jnp.where(valid, wv, _NEG_INF))
    wv9 = jnp.concatenate(wvs, axis=1)
    mx9 = _tree_cols(wv9, 9, jnp.maximum, _NEG_INF)
    e9 = jnp.exp(wv9 - mx9)
    den9 = _tree_cols(e9, 9, jnp.add, 0.0)
    wn = e9 / den9

    oh_all = jnp.concatenate(onehots, axis=0)
    fv_all = _exact_onehot_dot(oh_all, b_row)
    ff = fv_all[0:40, :] * wn[:, 0:1]
    for o in range(1, 9):
        ff = ff + fv_all[o * 40:(o + 1) * 40, :] * wn[:, o:o + 1]
    return jnp.where(m > 0, _exact_onehot_dot(p, ff), a_row)


def _body(x_ref, gs_ref, ps_ref, ms_ref, g20_ref, p20_ref, m20_ref,
          z_ref, o_ref):
    A = [x_ref[0, i] for i in range(4)]
    B = [x_ref[1, i] for i in range(4)]

    s6 = A[0] + A[1] + A[2] + B[0] + B[1] + B[2]
    mean = s6 / 6.0
    var = ((A[0] - mean) ** 2 + (A[1] - mean) ** 2 + (A[2] - mean) ** 2
           + (B[0] - mean) ** 2 + (B[1] - mean) ** 2 + (B[2] - mean) ** 2) / 5.0
    std = jnp.sqrt(var + 1e-8)

    def att(x, y, t):
        return _attention(x, y, gs_ref[0, t], ps_ref[0, t], ms_ref[0, t])

    A[0] = att(A[0], B[0], 0)
    B[1] = att(B[1], A[1], 2)
    A[2] = att(A[2], A[3], 4)
    B[2] = att(B[2], B[3], 6)
    B[0] = att(B[0], A[0], 1)
    A[1] = att(A[1], B[1], 3)
    A[3] = att(A[3], A[2], 5)
    B[3] = att(B[3], B[2], 7)

    mean_s = _split3(mean)
    std_s = _split3(std)
    order = ((A, 0), (B, 0), (A, 1), (B, 1), (A, 2), (B, 2), (A, 3), (B, 3))
    for mi, (half, i) in enumerate(order):
        g20 = g20_ref[0, mi]
        mean_sel = (_bdot(g20, mean_s[0]) + _bdot(g20, mean_s[1])) + _bdot(g20, mean_s[2])
        std_sel = (_bdot(g20, std_s[0]) + _bdot(g20, std_s[1])) + _bdot(g20, std_s[2])
        pts = mean_sel + std_sel * z_ref[0, mi]
        half[i] = jnp.where(m20_ref[0, mi] > 0,
                            _exact_onehot_dot(p20_ref[0, mi], pts), half[i])

    for i in range(4):
        o_ref[0, i] = A[i]
        o_ref[1, i] = B[i]


def kernel(inputss):
    x = inputss.reshape(2, _H, _P, _D)
    gs, ps, ms, g20, p20, m20, zsel = _CONSTS
    row_spec = pl.BlockSpec((2, 4, _P, _D), lambda b: (0, b, 0, 0))
    out = pl.pallas_call(
        _body,
        grid=(4,),
        in_specs=[
            row_spec,
            pl.BlockSpec((1, 8, 40, _P), lambda b: (b, 0, 0, 0)),
            pl.BlockSpec((1, 8, _P, 40), lambda b: (b, 0, 0, 0)),
            pl.BlockSpec((1, 8, _P, 1), lambda b: (b, 0, 0, 0)),
            pl.BlockSpec((1, 8, 20, _P), lambda b: (b, 0, 0, 0)),
            pl.BlockSpec((1, 8, _P, 20), lambda b: (b, 0, 0, 0)),
            pl.BlockSpec((1, 8, _P, 1), lambda b: (b, 0, 0, 0)),
            pl.BlockSpec((1, 8, 20, _D), lambda b: (b, 0, 0, 0)),
        ],
        out_specs=row_spec,
        out_shape=jax.ShapeDtypeStruct((2, _H, _P, _D), jnp.float32),
    )(x, gs, ps, ms, g20, p20, m20, zsel)
    return out.reshape(_N, _P, _D)


# speedup vs baseline: 1.7610x; 1.1291x over previous
import jax
import jax.numpy as jnp
import numpy as np
from jax.experimental import pallas as pl

_N, _P, _D = 32, 210, 768
_H = _N // 2
_NEG_INF = float("-inf")
_OFFS = (0, -1, 1, -10, 10, -11, -9, 9, 11)


def _build_constants():
    cpu = jax.local_devices(backend="cpu")[0]
    with jax.default_device(cpu):
        return _build_constants_impl()


def _build_constants_impl():
    base = jax.random.key(42)
    keys = jax.vmap(lambda c: jax.random.fold_in(base, c))(jnp.arange(1, 65))
    att_ids = jnp.asarray([16 * b + t for b in range(4) for t in range(8)])
    mut_ids = jnp.asarray([16 * b + 8 + t for b in range(4) for t in range(8)])
    att_keys = keys[att_ids]
    mut_keys = keys[mut_ids]
    att_perms = np.asarray(
        jax.vmap(lambda k: jax.random.permutation(k, _P))(att_keys))
    zs = np.asarray(
        jax.vmap(lambda k: jax.random.normal(k, (_P, _D), jnp.float32))(mut_keys))
    mut_perm_keys = jax.vmap(lambda k: jax.random.fold_in(k, 1))(mut_keys)
    mut_perms = np.asarray(
        jax.vmap(lambda k: jax.random.permutation(k, _P))(mut_perm_keys))

    r40 = att_perms[:, :40]
    gs = np.zeros((32, 40, _P), np.float32)
    gs[np.arange(32)[:, None], np.arange(40)[None, :], r40] = 1.0
    ps = np.ascontiguousarray(np.transpose(gs, (0, 2, 1)))
    ms = gs.sum(axis=1)[..., None]

    r20 = mut_perms[:, :20]
    g20 = np.zeros((32, 20, _P), np.float32)
    g20[np.arange(32)[:, None], np.arange(20)[None, :], r20] = 1.0
    p20 = np.ascontiguousarray(np.transpose(g20, (0, 2, 1)))
    m20 = g20.sum(axis=1)[..., None]
    zsel = zs[np.arange(32)[:, None], r20]

    return (
        gs.reshape(4, 8, 40, _P).astype(jnp.bfloat16),
        ps.reshape(4, 8, _P, 40).astype(jnp.bfloat16),
        ms.reshape(4, 8, _P, 1),
        g20.reshape(4, 8, 20, _P).astype(jnp.bfloat16),
        p20.reshape(4, 8, _P, 20).astype(jnp.bfloat16),
        m20.reshape(4, 8, _P, 1),
        zsel.reshape(4, 8, 20, _D),
    )


_CONSTS = _build_constants()


def _bdot(a_bf, b_bf):
    return jax.lax.dot_general(a_bf, b_bf, (((1,), (0,)), ((), ())),
                               preferred_element_type=jnp.float32)


def _split3(x):
    hi = x.astype(jnp.bfloat16)
    r1 = x - hi.astype(jnp.float32)
    lo1 = r1.astype(jnp.bfloat16)
    r2 = r1 - lo1.astype(jnp.float32)
    lo2 = r2.astype(jnp.bfloat16)
    return hi, lo1, lo2


def _exact_onehot_dot(oh_bf, x):
    hi, lo1, lo2 = _split3(x)
    return (_bdot(oh_bf, hi) + _bdot(oh_bf, lo1)) + _bdot(oh_bf, lo2)


def _tree_cols(e, n, op, pad_val):
    p2 = 1
    while p2 < n:
        p2 *= 2
    if p2 != n:
        pad = jnp.full((e.shape[0], p2 - n), pad_val, e.dtype)
        e = jnp.concatenate([e, pad], axis=1)
    stride = p2 // 2
    while stride >= 1:
        e = op(e[:, :stride], e[:, stride:2 * stride])
        stride //= 2
    return e


def _attention4(items):
    n = len(items)
    scores = [jax.lax.dot_general(
        a, b, (((1,), (1,)), ((), ())), preferred_element_type=jnp.float32)
        for a, b, _, _, _ in items]
    s_sel = [_exact_onehot_dot(items[i][2], scores[i]) for i in range(n)]

    ws = []
    for s in s_sel:
        mx = jnp.max(s, axis=1, keepdims=True)
        e = jnp.exp(s - mx)
        den = _tree_cols(e, _P, jnp.add, 0.0)
        ws.append(e / den)

    iota = jax.lax.broadcasted_iota(jnp.int32, (40, _P), 1)
    mps = []
    for w in ws:
        wmax = jnp.max(w, axis=1, keepdims=True)
        mps.append(jnp.min(jnp.where(w == wmax, iota, _P), axis=1,
                           keepdims=True))
    oh_alls, wns = [], []
    for w, mp in zip(ws, mps):
        col = mp % 10
        valids = (
            mp >= 0,
            col != 0,
            col != 9,
            mp > 9,
            mp < 200,
            (mp > 9) & (col > 0),
            (mp > 9) & (col < 9),
            (mp < 200) & (col > 0),
            (mp < 200) & (col < 9),
        )
        onehots, wvs = [], []
        for off, valid in zip(_OFFS, valids):
            idx = jnp.clip(mp + off, 0, _P - 1)
            oh = (iota == idx)
            onehots.append(oh.astype(jnp.bfloat16))
            wv = jnp.sum(oh.astype(jnp.float32) * w, axis=1, keepdims=True)
            wvs.append(jnp.where(valid, wv, _NEG_INF))
        wv9 = jnp.concatenate(wvs, axis=1)
        mx9 = _tree_cols(wv9, 9, jnp.maximum, _NEG_INF)
        e9 = jnp.exp(wv9 - mx9)
        den9 = _tree_cols(e9, 9, jnp.add, 0.0)
        wns.append(e9 / den9)
        oh_alls.append(jnp.concatenate(onehots, axis=0))

    fv_alls = [_exact_onehot_dot(oh_alls[i], items[i][1]) for i in range(n)]
    outs = []
    for i in range(n):
        wn, fv_all = wns[i], fv_alls[i]
        ff = fv_all[0:40, :] * wn[:, 0:1]
        for o in range(1, 9):
            ff = ff + fv_all[o * 40:(o + 1) * 40, :] * wn[:, o:o + 1]
        a_row, _, _, p, m = items[i]
        outs.append(jnp.where(m > 0, _exact_onehot_dot(p, ff), a_row))
    return outs


def _body(x_ref, gs_ref, ps_ref, ms_ref, g20_ref, p20_ref, m20_ref,
          z_ref, o_ref):
    A = [x_ref[0, i] for i in range(4)]
    B = [x_ref[1, i] for i in range(4)]

    s6 = A[0] + A[1] + A[2] + B[0] + B[1] + B[2]
    mean = s6 / 6.0
    var = ((A[0] - mean) ** 2 + (A[1] - mean) ** 2 + (A[2] - mean) ** 2
           + (B[0] - mean) ** 2 + (B[1] - mean) ** 2 + (B[2] - mean) ** 2) / 5.0
    std = jnp.sqrt(var + 1e-8)

    def item(x, y, t):
        return (x, y, gs_ref[0, t], ps_ref[0, t], ms_ref[0, t])

    A[0], B[1], A[2], B[2] = _attention4([
        item(A[0], B[0], 0),
        item(B[1], A[1], 2),
        item(A[2], A[3], 4),
        item(B[2], B[3], 6),
    ])
    B[0], A[1], A[3], B[3] = _attention4([
        item(B[0], A[0], 1),
        item(A[1], B[1], 3),
        item(A[3], A[2], 5),
        item(B[3], B[2], 7),
    ])

    mean_s = _split3(mean)
    std_s = _split3(std)
    order = ((A, 0), (B, 0), (A, 1), (B, 1), (A, 2), (B, 2), (A, 3), (B, 3))
    for mi, (half, i) in enumerate(order):
        g20 = g20_ref[0, mi]
        mean_sel = (_bdot(g20, mean_s[0]) + _bdot(g20, mean_s[1])) + _bdot(g20, mean_s[2])
        std_sel = (_bdot(g20, std_s[0]) + _bdot(g20, std_s[1])) + _bdot(g20, std_s[2])
        pts = mean_sel + std_sel * z_ref[0, mi]
        half[i] = jnp.where(m20_ref[0, mi] > 0,
                            _exact_onehot_dot(p20_ref[0, mi], pts), half[i])

    for i in range(4):
        o_ref[0, i] = A[i]
        o_ref[1, i] = B[i]


def kernel(inputss):
    x = inputss.reshape(2, _H, _P, _D)
    gs, ps, ms, g20, p20, m20, zsel = _CONSTS
    row_spec = pl.BlockSpec((2, 4, _P, _D), lambda b: (0, b, 0, 0))
    out = pl.pallas_call(
        _body,
        grid=(4,),
        in_specs=[
            row_spec,
            pl.BlockSpec((1, 8, 40, _P), lambda b: (b, 0, 0, 0)),
            pl.BlockSpec((1, 8, _P, 40), lambda b: (b, 0, 0, 0)),
            pl.BlockSpec((1, 8, _P, 1), lambda b: (b, 0, 0, 0)),
            pl.BlockSpec((1, 8, 20, _P), lambda b: (b, 0, 0, 0)),
            pl.BlockSpec((1, 8, _P, 20), lambda b: (b, 0, 0, 0)),
            pl.BlockSpec((1, 8, _P, 1), lambda b: (b, 0, 0, 0)),
            pl.BlockSpec((1, 8, 20, _D), lambda b: (b, 0, 0, 0)),
        ],
        out_specs=row_spec,
        out_shape=jax.ShapeDtypeStruct((2, _H, _P, _D), jnp.float32),
    )(x, gs, ps, ms, g20, p20, m20, zsel)
    return out.reshape(_N, _P, _D)


# speedup vs baseline: 1.9170x; 1.0886x over previous
import jax
import jax.numpy as jnp
import numpy as np
from jax.experimental import pallas as pl

_N, _P, _D = 32, 210, 768
_H = _N // 2
_NEG_INF = float("-inf")
_OFFS = (0, -1, 1, -10, 10, -11, -9, 9, 11)


def _build_constants():
    cpu = jax.local_devices(backend="cpu")[0]
    with jax.default_device(cpu):
        return _build_constants_impl()


def _build_constants_impl():
    base = jax.random.key(42)
    keys = jax.vmap(lambda c: jax.random.fold_in(base, c))(jnp.arange(1, 65))
    att_ids = jnp.asarray([16 * b + t for b in range(4) for t in range(8)])
    mut_ids = jnp.asarray([16 * b + 8 + t for b in range(4) for t in range(8)])
    att_keys = keys[att_ids]
    mut_keys = keys[mut_ids]
    att_perms = np.asarray(
        jax.vmap(lambda k: jax.random.permutation(k, _P))(att_keys))
    zs = np.asarray(
        jax.vmap(lambda k: jax.random.normal(k, (_P, _D), jnp.float32))(mut_keys))
    mut_perm_keys = jax.vmap(lambda k: jax.random.fold_in(k, 1))(mut_keys)
    mut_perms = np.asarray(
        jax.vmap(lambda k: jax.random.permutation(k, _P))(mut_perm_keys))

    r40 = att_perms[:, :40]
    gs = np.zeros((32, 40, _P), np.float32)
    gs[np.arange(32)[:, None], np.arange(40)[None, :], r40] = 1.0
    ps = np.ascontiguousarray(np.transpose(gs, (0, 2, 1)))
    ms = gs.sum(axis=1)[..., None]

    r20 = mut_perms[:, :20]
    g20 = np.zeros((32, 20, _P), np.float32)
    g20[np.arange(32)[:, None], np.arange(20)[None, :], r20] = 1.0
    p20 = np.ascontiguousarray(np.transpose(g20, (0, 2, 1)))
    m20 = g20.sum(axis=1)[..., None]
    zsel = zs[np.arange(32)[:, None], r20]

    return (
        gs.reshape(4, 8, 40, _P).astype(jnp.bfloat16),
        ps.reshape(4, 8, _P, 40).astype(jnp.bfloat16),
        ms.reshape(4, 8, _P, 1),
        g20.reshape(4, 8, 20, _P).astype(jnp.bfloat16),
        p20.reshape(4, 8, _P, 20).astype(jnp.bfloat16),
        m20.reshape(4, 8, _P, 1),
        zsel.reshape(4, 8, 20, _D),
    )


_CONSTS = _build_constants()


def _bdot(a_bf, b_bf):
    return jax.lax.dot_general(a_bf, b_bf, (((1,), (0,)), ((), ())),
                               preferred_element_type=jnp.float32)


def _split3(x):
    hi = x.astype(jnp.bfloat16)
    r1 = x - hi.astype(jnp.float32)
    lo1 = r1.astype(jnp.bfloat16)
    r2 = r1 - lo1.astype(jnp.float32)
    lo2 = r2.astype(jnp.bfloat16)
    return hi, lo1, lo2


def _exact_onehot_dot(oh_bf, x):
    hi, lo1, lo2 = _split3(x)
    return (_bdot(oh_bf, hi) + _bdot(oh_bf, lo1)) + _bdot(oh_bf, lo2)


def _tree_cols(e, n, op, pad_val):
    p2 = 1
    while p2 < n:
        p2 *= 2
    if p2 != n:
        pad = jnp.full((e.shape[0], p2 - n), pad_val, e.dtype)
        e = jnp.concatenate([e, pad], axis=1)
    stride = p2 // 2
    while stride >= 1:
        e = op(e[:, :stride], e[:, stride:2 * stride])
        stride //= 2
    return e


def _attention4(items):
    n = len(items)
    scores = [jax.lax.dot_general(
        a, b, (((1,), (1,)), ((), ())), preferred_element_type=jnp.float32)
        for a, b, _, _, _ in items]
    s_sel = [_exact_onehot_dot(items[i][2], scores[i]) for i in range(n)]

    ws = []
    for s in s_sel:
        mx = jnp.max(s, axis=1, keepdims=True)
        e = jnp.exp(s - mx)
        den = _tree_cols(e, _P, jnp.add, 0.0)
        ws.append(e / den)

    iota = jax.lax.broadcasted_iota(jnp.int32, (40, _P), 1)
    mps = []
    for w in ws:
        wmax = jnp.max(w, axis=1, keepdims=True)
        mps.append(jnp.min(jnp.where(w == wmax, iota, _P), axis=1,
                           keepdims=True))
    oh_alls, wns = [], []
    for w, mp in zip(ws, mps):
        col = mp % 10
        valids = (
            mp >= 0,
            col != 0,
            col != 9,
            mp > 9,
            mp < 200,
            (mp > 9) & (col > 0),
            (mp > 9) & (col < 9),
            (mp < 200) & (col > 0),
            (mp < 200) & (col < 9),
        )
        onehots, wvs = [], []
        for off, valid in zip(_OFFS, valids):
            idx = jnp.clip(mp + off, 0, _P - 1)
            oh = (iota == idx)
            onehots.append(oh.astype(jnp.bfloat16))
            wv = jnp.sum(oh.astype(jnp.float32) * w, axis=1, keepdims=True)
            wvs.append(jnp.where(valid, wv, _NEG_INF))
        wv9 = jnp.concatenate(wvs, axis=1)
        mx9 = _tree_cols(wv9, 9, jnp.maximum, _NEG_INF)
        e9 = jnp.exp(wv9 - mx9)
        den9 = _tree_cols(e9, 9, jnp.add, 0.0)
        wns.append(e9 / den9)
        oh_alls.append(jnp.concatenate(onehots, axis=0))

    fv_alls = [_exact_onehot_dot(oh_alls[i], items[i][1]) for i in range(n)]
    outs = []
    for i in range(n):
        wn, fv_all = wns[i], fv_alls[i]
        ff = fv_all[0:40, :] * wn[:, 0:1]
        for o in range(1, 9):
            ff = ff + fv_all[o * 40:(o + 1) * 40, :] * wn[:, o:o + 1]
        a_row, _, _, p, m = items[i]
        outs.append(jnp.where(m > 0, _exact_onehot_dot(p, ff), a_row))
    return outs


def _body(x_ref, gs_ref, ps_ref, ms_ref, g20_ref, p20_ref, m20_ref,
          z_ref, o_ref):
    A = [x_ref[0, i] for i in range(4)]
    B = [x_ref[1, i] for i in range(4)]

    s6 = A[0] + A[1] + A[2] + B[0] + B[1] + B[2]
    mean = s6 / 6.0
    var = ((A[0] - mean) ** 2 + (A[1] - mean) ** 2 + (A[2] - mean) ** 2
           + (B[0] - mean) ** 2 + (B[1] - mean) ** 2 + (B[2] - mean) ** 2) / 5.0
    std = jnp.sqrt(var + 1e-8)

    def item(x, y, t):
        return (x, y, gs_ref[0, t], ps_ref[0, t], ms_ref[0, t])

    A[0], B[1], A[2], B[2] = _attention4([
        item(A[0], B[0], 0),
        item(B[1], A[1], 2),
        item(A[2], A[3], 4),
        item(B[2], B[3], 6),
    ])
    B[0], A[1], A[3], B[3] = _attention4([
        item(B[0], A[0], 1),
        item(A[1], B[1], 3),
        item(A[3], A[2], 5),
        item(B[3], B[2], 7),
    ])

    g20_all = g20_ref[0].reshape(8 * 20, _P)
    z_all = z_ref[0].reshape(8 * 20, _D)
    mean_sel = _exact_onehot_dot(g20_all, mean)
    std_sel = _exact_onehot_dot(g20_all, std)
    pts_all = mean_sel + std_sel * z_all
    order = ((A, 0), (B, 0), (A, 1), (B, 1), (A, 2), (B, 2), (A, 3), (B, 3))
    for mi, (half, i) in enumerate(order):
        pts = pts_all[mi * 20:(mi + 1) * 20, :]
        half[i] = jnp.where(m20_ref[0, mi] > 0,
                            _exact_onehot_dot(p20_ref[0, mi], pts), half[i])

    for i in range(4):
        o_ref[0, i] = A[i]
        o_ref[1, i] = B[i]


def kernel(inputss):
    x = inputss.reshape(2, _H, _P, _D)
    gs, ps, ms, g20, p20, m20, zsel = _CONSTS
    row_spec = pl.BlockSpec((2, 4, _P, _D), lambda b: (0, b, 0, 0))
    out = pl.pallas_call(
        _body,
        grid=(4,),
        in_specs=[
            row_spec,
            pl.BlockSpec((1, 8, 40, _P), lambda b: (b, 0, 0, 0)),
            pl.BlockSpec((1, 8, _P, 40), lambda b: (b, 0, 0, 0)),
            pl.BlockSpec((1, 8, _P, 1), lambda b: (b, 0, 0, 0)),
            pl.BlockSpec((1, 8, 20, _P), lambda b: (b, 0, 0, 0)),
            pl.BlockSpec((1, 8, _P, 20), lambda b: (b, 0, 0, 0)),
            pl.BlockSpec((1, 8, _P, 1), lambda b: (b, 0, 0, 0)),
            pl.BlockSpec((1, 8, 20, _D), lambda b: (b, 0, 0, 0)),
        ],
        out_specs=row_spec,
        out_shape=jax.ShapeDtypeStruct((2, _H, _P, _D), jnp.float32),
    )(x, gs, ps, ms, g20, p20, m20, zsel)
    return out.reshape(_N, _P, _D)


# speedup vs baseline: 1.9411x; 1.0126x over previous
import jax
import jax.numpy as jnp
import numpy as np
from jax.experimental import pallas as pl

_N, _P, _D = 32, 210, 768
_H = _N // 2
_NEG_INF = float("-inf")
_OFFS = (0, -1, 1, -10, 10, -11, -9, 9, 11)


def _build_constants():
    cpu = jax.local_devices(backend="cpu")[0]
    with jax.default_device(cpu):
        return _build_constants_impl()


def _build_constants_impl():
    base = jax.random.key(42)
    keys = jax.vmap(lambda c: jax.random.fold_in(base, c))(jnp.arange(1, 65))
    att_ids = jnp.asarray([16 * b + t for b in range(4) for t in range(8)])
    mut_ids = jnp.asarray([16 * b + 8 + t for b in range(4) for t in range(8)])
    att_keys = keys[att_ids]
    mut_keys = keys[mut_ids]
    att_perms = np.asarray(
        jax.vmap(lambda k: jax.random.permutation(k, _P))(att_keys))
    zs = np.asarray(
        jax.vmap(lambda k: jax.random.normal(k, (_P, _D), jnp.float32))(mut_keys))
    mut_perm_keys = jax.vmap(lambda k: jax.random.fold_in(k, 1))(mut_keys)
    mut_perms = np.asarray(
        jax.vmap(lambda k: jax.random.permutation(k, _P))(mut_perm_keys))

    r40 = att_perms[:, :40]
    gs = np.zeros((32, 40, _P), np.float32)
    gs[np.arange(32)[:, None], np.arange(40)[None, :], r40] = 1.0
    ms = gs.sum(axis=1)[..., None]

    gblk = np.zeros((4, 2, 160, 4 * _P), np.float32)
    pblk = np.zeros((4, 2, 4 * _P, 160), np.float32)
    for b in range(4):
        for grp in range(2):
            for j in range(4):
                t = 2 * j + grp
                ridx = r40[8 * b + t]
                rows = j * 40 + np.arange(40)
                cols = j * _P + ridx
                gblk[b, grp, rows, cols] = 1.0
                pblk[b, grp, cols, rows] = 1.0

    r20 = mut_perms[:, :20]
    g20 = np.zeros((32, 20, _P), np.float32)
    g20[np.arange(32)[:, None], np.arange(20)[None, :], r20] = 1.0
    p20 = np.ascontiguousarray(np.transpose(g20, (0, 2, 1)))
    m20 = g20.sum(axis=1)[..., None]
    zsel = zs[np.arange(32)[:, None], r20]

    return (
        gblk.astype(jnp.bfloat16),
        pblk.astype(jnp.bfloat16),
        ms.reshape(4, 8, _P, 1),
        g20.reshape(4, 8, 20, _P).astype(jnp.bfloat16),
        p20.reshape(4, 8, _P, 20).astype(jnp.bfloat16),
        m20.reshape(4, 8, _P, 1),
        zsel.reshape(4, 8, 20, _D),
    )


_CONSTS = _build_constants()


def _bdot(a_bf, b_bf):
    return jax.lax.dot_general(a_bf, b_bf, (((1,), (0,)), ((), ())),
                               preferred_element_type=jnp.float32)


def _split3(x):
    hi = x.astype(jnp.bfloat16)
    r1 = x - hi.astype(jnp.float32)
    lo1 = r1.astype(jnp.bfloat16)
    r2 = r1 - lo1.astype(jnp.float32)
    lo2 = r2.astype(jnp.bfloat16)
    return hi, lo1, lo2


def _exact_onehot_dot(oh_bf, x):
    hi, lo1, lo2 = _split3(x)
    return (_bdot(oh_bf, hi) + _bdot(oh_bf, lo1)) + _bdot(oh_bf, lo2)


def _tree_cols(e, n, op, pad_val):
    p2 = 1
    while p2 < n:
        p2 *= 2
    if p2 != n:
        pad = jnp.full((e.shape[0], p2 - n), pad_val, e.dtype)
        e = jnp.concatenate([e, pad], axis=1)
    stride = p2 // 2
    while stride >= 1:
        e = op(e[:, :stride], e[:, stride:2 * stride])
        stride //= 2
    return e


def _attention4(items, gblk, pblk):
    n = len(items)
    scores = [jax.lax.dot_general(
        a, b, (((1,), (1,)), ((), ())), preferred_element_type=jnp.float32)
        for a, b, _ in items]
    s_sel_all = _exact_onehot_dot(gblk, jnp.concatenate(scores, axis=0))
    s_sel = [s_sel_all[i * 40:(i + 1) * 40, :] for i in range(n)]

    ws = []
    for s in s_sel:
        mx = jnp.max(s, axis=1, keepdims=True)
        e = jnp.exp(s - mx)
        den = _tree_cols(e, _P, jnp.add, 0.0)
        ws.append(e / den)

    iota = jax.lax.broadcasted_iota(jnp.int32, (40, _P), 1)
    mps = []
    for w in ws:
        wmax = jnp.max(w, axis=1, keepdims=True)
        mps.append(jnp.min(jnp.where(w == wmax, iota, _P), axis=1,
                           keepdims=True))
    oh_alls, wns = [], []
    for w, mp in zip(ws, mps):
        col = mp % 10
        valids = (
            mp >= 0,
            col != 0,
            col != 9,
            mp > 9,
            mp < 200,
            (mp > 9) & (col > 0),
            (mp > 9) & (col < 9),
            (mp < 200) & (col > 0),
            (mp < 200) & (col < 9),
        )
        onehots, wvs = [], []
        for off, valid in zip(_OFFS, valids):
            idx = jnp.clip(mp + off, 0, _P - 1)
            oh = (iota == idx)
            onehots.append(oh.astype(jnp.bfloat16))
            wv = jnp.sum(oh.astype(jnp.float32) * w, axis=1, keepdims=True)
            wvs.append(jnp.where(valid, wv, _NEG_INF))
        wv9 = jnp.concatenate(wvs, axis=1)
        mx9 = _tree_cols(wv9, 9, jnp.maximum, _NEG_INF)
        e9 = jnp.exp(wv9 - mx9)
        den9 = _tree_cols(e9, 9, jnp.add, 0.0)
        wns.append(e9 / den9)
        oh_alls.append(jnp.concatenate(onehots, axis=0))

    fv_alls = [_exact_onehot_dot(oh_alls[i], items[i][1]) for i in range(n)]
    ffs = []
    for i in range(n):
        wn, fv_all = wns[i], fv_alls[i]
        ff = fv_all[0:40, :] * wn[:, 0:1]
        for o in range(1, 9):
            ff = ff + fv_all[o * 40:(o + 1) * 40, :] * wn[:, o:o + 1]
        ffs.append(ff)
    scat_all = _exact_onehot_dot(pblk, jnp.concatenate(ffs, axis=0))
    outs = []
    for i in range(n):
        a_row, _, m = items[i]
        outs.append(jnp.where(m > 0, scat_all[i * _P:(i + 1) * _P, :], a_row))
    return outs


def _body(x_ref, gs_ref, ps_ref, ms_ref, g20_ref, p20_ref, m20_ref,
          z_ref, o_ref):
    A = [x_ref[0, i] for i in range(4)]
    B = [x_ref[1, i] for i in range(4)]

    s6 = A[0] + A[1] + A[2] + B[0] + B[1] + B[2]
    mean = s6 / 6.0
    var = ((A[0] - mean) ** 2 + (A[1] - mean) ** 2 + (A[2] - mean) ** 2
           + (B[0] - mean) ** 2 + (B[1] - mean) ** 2 + (B[2] - mean) ** 2) / 5.0
    std = jnp.sqrt(var + 1e-8)

    def item(x, y, t):
        return (x, y, ms_ref[0, t])

    A[0], B[1], A[2], B[2] = _attention4([
        item(A[0], B[0], 0),
        item(B[1], A[1], 2),
        item(A[2], A[3], 4),
        item(B[2], B[3], 6),
    ], gs_ref[0, 0], ps_ref[0, 0])
    B[0], A[1], A[3], B[3] = _attention4([
        item(B[0], A[0], 1),
        item(A[1], B[1], 3),
        item(A[3], A[2], 5),
        item(B[3], B[2], 7),
    ], gs_ref[0, 1], ps_ref[0, 1])

    g20_all = g20_ref[0].reshape(8 * 20, _P)
    z_all = z_ref[0].reshape(8 * 20, _D)
    mean_sel = _exact_onehot_dot(g20_all, mean)
    std_sel = _exact_onehot_dot(g20_all, std)
    pts_all = mean_sel + std_sel * z_all
    order = ((A, 0), (B, 0), (A, 1), (B, 1), (A, 2), (B, 2), (A, 3), (B, 3))
    for mi, (half, i) in enumerate(order):
        pts = pts_all[mi * 20:(mi + 1) * 20, :]
        half[i] = jnp.where(m20_ref[0, mi] > 0,
                            _exact_onehot_dot(p20_ref[0, mi], pts), half[i])

    for i in range(4):
        o_ref[0, i] = A[i]
        o_ref[1, i] = B[i]


def kernel(inputss):
    x = inputss.reshape(2, _H, _P, _D)
    gs, ps, ms, g20, p20, m20, zsel = _CONSTS
    row_spec = pl.BlockSpec((2, 4, _P, _D), lambda b: (0, b, 0, 0))
    out = pl.pallas_call(
        _body,
        grid=(4,),
        in_specs=[
            row_spec,
            pl.BlockSpec((1, 2, 160, 4 * _P), lambda b: (b, 0, 0, 0)),
            pl.BlockSpec((1, 2, 4 * _P, 160), lambda b: (b, 0, 0, 0)),
            pl.BlockSpec((1, 8, _P, 1), lambda b: (b, 0, 0, 0)),
            pl.BlockSpec((1, 8, 20, _P), lambda b: (b, 0, 0, 0)),
            pl.BlockSpec((1, 8, _P, 20), lambda b: (b, 0, 0, 0)),
            pl.BlockSpec((1, 8, _P, 1), lambda b: (b, 0, 0, 0)),
            pl.BlockSpec((1, 8, 20, _D), lambda b: (b, 0, 0, 0)),
        ],
        out_specs=row_spec,
        out_shape=jax.ShapeDtypeStruct((2, _H, _P, _D), jnp.float32),
    )(x, gs, ps, ms, g20, p20, m20, zsel)
    return out.reshape(_N, _P, _D)
